# initial kernel scaffold (unmeasured)
import jax
import jax.numpy as jnp
from jax import lax
from jax.experimental import pallas as pl
from jax.experimental.pallas import tpu as pltpu


def kernel(
    x,
):
    def body(*refs):
        pass

    out_shape = jax.ShapeDtypeStruct(..., jnp.float32)
    return pl.pallas_call(body, out_shape=out_shape)(...)


# baseline (device time: 31425 ns/iter reference)
import jax
import jax.numpy as jnp
from jax import lax
from jax.experimental import pallas as pl
from jax.experimental.pallas import tpu as pltpu

N_DEV = 8


def _bitonic_sort(v, n_rows):
    row = lax.broadcasted_iota(jnp.int32, v.shape, 0)
    k = 2
    while k <= n_rows:
        j = k // 2
        while j >= 1:
            down = jnp.roll(v, -j, axis=0)
            up = jnp.roll(v, j, axis=0)
            lower = (row & j) == 0
            partner = jnp.where(lower, down, up)
            asc = (row & k) == 0
            take_min = lower == asc
            v = jnp.where(take_min, jnp.minimum(v, partner),
                          jnp.maximum(v, partner))
            j //= 2
        k *= 2
    return v


def kernel(x):
    m_per, n = x.shape
    n_total = N_DEV * m_per

    def body(x_ref, out_ref, gbuf_ref, send_sems, recv_sems):
        my = lax.axis_index("i")
        left = (my - 1) % N_DEV
        right = (my + 1) % N_DEV

        gbuf_ref[pl.ds(my * m_per, m_per), :] = x_ref[:, :].astype(jnp.bfloat16)

        barrier_sem = pltpu.get_barrier_semaphore()
        for nbr in (left, right):
            pl.semaphore_signal(
                barrier_sem, inc=1,
                device_id=(nbr,), device_id_type=pl.DeviceIdType.MESH,
            )
        pl.semaphore_wait(barrier_sem, 2)

        for h in range(N_DEV - 1):
            origin = (my - h) % N_DEV
            slot = gbuf_ref.at[pl.ds(origin * m_per, m_per), :]
            rdma = pltpu.make_async_remote_copy(
                src_ref=slot,
                dst_ref=slot,
                send_sem=send_sems.at[h],
                recv_sem=recv_sems.at[h],
                device_id=(right,),
                device_id_type=pl.DeviceIdType.MESH,
            )
            rdma.start()
            rdma.wait()

        v = _bitonic_sort(gbuf_ref[:, :], n_total)

        gbuf_ref[:, :] = v
        out_ref[:, :] = gbuf_ref[pl.ds(my * m_per, m_per), :].astype(jnp.float32)

    return pl.pallas_call(
        body,
        out_shape=jax.ShapeDtypeStruct((m_per, n), jnp.float32),
        in_specs=[pl.BlockSpec(memory_space=pltpu.VMEM)],
        out_specs=pl.BlockSpec(memory_space=pltpu.VMEM),
        scratch_shapes=[
            pltpu.VMEM((n_total, n), jnp.bfloat16),
            pltpu.SemaphoreType.DMA((N_DEV - 1,)),
            pltpu.SemaphoreType.DMA((N_DEV - 1,)),
        ],
        compiler_params=pltpu.CompilerParams(collective_id=0),
    )(x)


# device time: 15131 ns/iter; 2.0769x vs baseline; 2.0769x over previous
import jax
import jax.numpy as jnp
from jax import lax
from jax.experimental import pallas as pl
from jax.experimental.pallas import tpu as pltpu

N_DEV = 8


def _bitonic_stages(v, k_first, k_last, dir_asc=None):
    row = lax.broadcasted_iota(jnp.int32, v.shape, 0)
    k = k_first
    while k <= k_last:
        asc = (row & k) == 0
        if dir_asc is not None:
            asc = asc == dir_asc
        j = k // 2
        while j >= 1:
            down = jnp.roll(v, -j, axis=0)
            up = jnp.roll(v, j, axis=0)
            lower = (row & j) == 0
            partner = jnp.where(lower, down, up)
            take_min = lower == asc
            v = jnp.where(take_min, jnp.minimum(v, partner),
                          jnp.maximum(v, partner))
            j //= 2
        k *= 2
    return v


def kernel(x):
    m_per, n = x.shape
    n_total = N_DEV * m_per

    def body(x_ref, out_ref, gbuf_ref, send_sems, recv_sems):
        my = lax.axis_index("i")

        dir_asc = (my % 2) == 0
        v_loc = _bitonic_stages(x_ref[:, :].astype(jnp.bfloat16),
                                2, m_per, dir_asc=dir_asc)
        gbuf_ref[pl.ds(my * m_per, m_per), :] = v_loc

        barrier_sem = pltpu.get_barrier_semaphore()
        for off in range(1, N_DEV):
            pl.semaphore_signal(
                barrier_sem, inc=1,
                device_id=((my + off) % N_DEV,),
                device_id_type=pl.DeviceIdType.MESH,
            )
        pl.semaphore_wait(barrier_sem, N_DEV - 1)

        my_slot = gbuf_ref.at[pl.ds(my * m_per, m_per), :]
        sends = []
        for off in range(1, N_DEV):
            rdma = pltpu.make_async_remote_copy(
                src_ref=my_slot,
                dst_ref=my_slot,
                send_sem=send_sems.at[off - 1],
                recv_sem=recv_sems.at[my],
                device_id=((my + off) % N_DEV,),
                device_id_type=pl.DeviceIdType.MESH,
            )
            rdma.start()
            sends.append(rdma)

        for off in range(1, N_DEV):
            src_dev = (my + off) % N_DEV
            slot = gbuf_ref.at[pl.ds(src_dev * m_per, m_per), :]
            recv = pltpu.make_async_remote_copy(
                src_ref=slot,
                dst_ref=slot,
                send_sem=send_sems.at[off - 1],
                recv_sem=recv_sems.at[src_dev],
                device_id=(src_dev,),
                device_id_type=pl.DeviceIdType.MESH,
            )
            recv.wait_recv()

        v = _bitonic_stages(gbuf_ref[:, :], 2 * m_per, n_total)

        gbuf_ref[:, :] = v
        out_ref[:, :] = gbuf_ref[pl.ds(my * m_per, m_per), :].astype(jnp.float32)

        for rdma in sends:
            rdma.wait_send()

    return pl.pallas_call(
        body,
        out_shape=jax.ShapeDtypeStruct((m_per, n), jnp.float32),
        in_specs=[pl.BlockSpec(memory_space=pltpu.VMEM)],
        out_specs=pl.BlockSpec(memory_space=pltpu.VMEM),
        scratch_shapes=[
            pltpu.VMEM((n_total, n), jnp.bfloat16),
            pltpu.SemaphoreType.DMA((N_DEV - 1,)),
            pltpu.SemaphoreType.DMA((N_DEV,)),
        ],
        compiler_params=pltpu.CompilerParams(collective_id=0),
    )(x)


# device time: 12689 ns/iter; 2.4766x vs baseline; 1.1925x over previous
import jax
import jax.numpy as jnp
from jax import lax
from jax.experimental import pallas as pl
from jax.experimental.pallas import tpu as pltpu

N_DEV = 8


def _pack(v):
    r = v.shape[0]
    return jnp.concatenate([v[: r // 2], v[r // 2 :]], axis=1)


def _unpack(v2):
    c = v2.shape[1] // 2
    return jnp.concatenate([v2[:, :c], v2[:, c:]], axis=0)


def _bitonic_stages_packed(v2, k_first, k_last, dir_asc=None):
    r2, c2 = v2.shape
    c = c2 // 2
    lane = lax.broadcasted_iota(jnp.int32, v2.shape, 1)
    row = lax.broadcasted_iota(jnp.int32, v2.shape, 0) + jnp.where(
        lane >= c, r2, 0
    )
    k = k_first
    while k <= k_last:
        asc = (row & k) == 0
        if dir_asc is not None:
            asc = asc == dir_asc
        j = k // 2
        while j >= 1:
            lower = (row & j) == 0
            if j == r2:
                partner = jnp.roll(v2, c, axis=1)
            else:
                down = jnp.roll(v2, -j, axis=0)
                up = jnp.roll(v2, j, axis=0)
                partner = jnp.where(lower, down, up)
            take_min = lower == asc
            v2 = jnp.where(take_min, jnp.minimum(v2, partner),
                           jnp.maximum(v2, partner))
            j //= 2
        k *= 2
    return v2


def kernel(x):
    m_per, n = x.shape
    n_total = N_DEV * m_per

    def body(x_ref, out_ref, gbuf_ref, send_sems, recv_sems):
        my = lax.axis_index("i")

        barrier_sem = pltpu.get_barrier_semaphore()
        for off in range(1, N_DEV):
            pl.semaphore_signal(
                barrier_sem, inc=1,
                device_id=((my + off) % N_DEV,),
                device_id_type=pl.DeviceIdType.MESH,
            )

        dir_asc = (my % 2) == 0
        v_loc = _bitonic_stages_packed(
            _pack(x_ref[:, :].astype(jnp.bfloat16)), 2, m_per, dir_asc=dir_asc
        )
        gbuf_ref[pl.ds(my * m_per, m_per), :] = _unpack(v_loc)

        pl.semaphore_wait(barrier_sem, N_DEV - 1)

        my_slot = gbuf_ref.at[pl.ds(my * m_per, m_per), :]
        sends = []
        for off in range(1, N_DEV):
            rdma = pltpu.make_async_remote_copy(
                src_ref=my_slot,
                dst_ref=my_slot,
                send_sem=send_sems.at[off - 1],
                recv_sem=recv_sems.at[my],
                device_id=((my + off) % N_DEV,),
                device_id_type=pl.DeviceIdType.MESH,
            )
            rdma.start()
            sends.append(rdma)

        for off in range(1, N_DEV):
            src_dev = (my + off) % N_DEV
            slot = gbuf_ref.at[pl.ds(src_dev * m_per, m_per), :]
            recv = pltpu.make_async_remote_copy(
                src_ref=slot,
                dst_ref=slot,
                send_sem=send_sems.at[off - 1],
                recv_sem=recv_sems.at[src_dev],
                device_id=(src_dev,),
                device_id_type=pl.DeviceIdType.MESH,
            )
            recv.wait_recv()

        v2 = _bitonic_stages_packed(_pack(gbuf_ref[:, :]), 2 * m_per, n_total)

        gbuf_ref[:, :] = _unpack(v2)
        out_ref[:, :] = gbuf_ref[pl.ds(my * m_per, m_per), :].astype(jnp.float32)

        for rdma in sends:
            rdma.wait_send()

    return pl.pallas_call(
        body,
        out_shape=jax.ShapeDtypeStruct((m_per, n), jnp.float32),
        in_specs=[pl.BlockSpec(memory_space=pltpu.VMEM)],
        out_specs=pl.BlockSpec(memory_space=pltpu.VMEM),
        scratch_shapes=[
            pltpu.VMEM((n_total, n), jnp.bfloat16),
            pltpu.SemaphoreType.DMA((N_DEV - 1,)),
            pltpu.SemaphoreType.DMA((N_DEV,)),
        ],
        compiler_params=pltpu.CompilerParams(collective_id=0),
    )(x)


# device time: 9362 ns/iter; 3.3567x vs baseline; 1.3554x over previous
import os

import jax
import jax.numpy as jnp
from jax import lax
from jax.experimental import pallas as pl
from jax.experimental.pallas import tpu as pltpu

N_DEV = 8
_ABLATE = os.environ.get("ABLATE", "")


def _pack(v):
    r = v.shape[0]
    return jnp.concatenate([v[: r // 2], v[r // 2 :]], axis=1)


def _unpack(v2):
    c = v2.shape[1] // 2
    return jnp.concatenate([v2[:, :c], v2[:, c:]], axis=0)


def _bitonic_stages_packed(v2, k_first, k_last, dir_asc=None):
    r2, c2 = v2.shape
    c = c2 // 2
    lane = lax.broadcasted_iota(jnp.int32, v2.shape, 1)
    row = lax.broadcasted_iota(jnp.int32, v2.shape, 0) + jnp.where(
        lane >= c, r2, 0
    )
    k = k_first
    while k <= k_last:
        asc = (row & k) == 0
        if dir_asc is not None:
            asc = asc == dir_asc
        j = k // 2
        while j >= 1:
            lower = (row & j) == 0
            if j == r2:
                partner = jnp.roll(v2, c, axis=1)
            else:
                down = jnp.roll(v2, -j, axis=0)
                up = jnp.roll(v2, j, axis=0)
                partner = jnp.where(lower, down, up)
            take_min = lower == asc
            v2 = jnp.where(take_min, jnp.minimum(v2, partner),
                           jnp.maximum(v2, partner))
            j //= 2
        k *= 2
    return v2


def kernel(x):
    m_per, n = x.shape
    n_total = N_DEV * m_per

    def body(x_ref, out_ref, gbuf_ref, send_sems, recv_sems):
        my = lax.axis_index("i")

        barrier_sem = pltpu.get_barrier_semaphore()
        for off in range(1, N_DEV):
            pl.semaphore_signal(
                barrier_sem, inc=1,
                device_id=((my + off) % N_DEV,),
                device_id_type=pl.DeviceIdType.MESH,
            )

        dir_asc = (my % 2) == 0
        if _ABLATE in ("nosort", "nolocal"):
            gbuf_ref[pl.ds(my * m_per, m_per), :] = x_ref[:, :].astype(jnp.bfloat16)
        else:
            v_loc = _bitonic_stages_packed(
                _pack(x_ref[:, :].astype(jnp.bfloat16)), 2, m_per, dir_asc=dir_asc
            )
            gbuf_ref[pl.ds(my * m_per, m_per), :] = _unpack(v_loc)

        pl.semaphore_wait(barrier_sem, N_DEV - 1)

        my_slot = gbuf_ref.at[pl.ds(my * m_per, m_per), :]
        sends = []
        for off in range(1, N_DEV):
            rdma = pltpu.make_async_remote_copy(
                src_ref=my_slot,
                dst_ref=my_slot,
                send_sem=send_sems.at[off - 1],
                recv_sem=recv_sems.at[my],
                device_id=((my + off) % N_DEV,),
                device_id_type=pl.DeviceIdType.MESH,
            )
            rdma.start()
            sends.append(rdma)

        for off in range(1, N_DEV):
            src_dev = (my + off) % N_DEV
            slot = gbuf_ref.at[pl.ds(src_dev * m_per, m_per), :]
            recv = pltpu.make_async_remote_copy(
                src_ref=slot,
                dst_ref=slot,
                send_sem=send_sems.at[off - 1],
                recv_sem=recv_sems.at[src_dev],
                device_id=(src_dev,),
                device_id_type=pl.DeviceIdType.MESH,
            )
            recv.wait_recv()

        if _ABLATE not in ("nosort", "nomerge"):
            v2 = _bitonic_stages_packed(_pack(gbuf_ref[:, :]), 2 * m_per, n_total)
            gbuf_ref[:, :] = _unpack(v2)
        out_ref[:, :] = gbuf_ref[pl.ds(my * m_per, m_per), :].astype(jnp.float32)

        for rdma in sends:
            rdma.wait_send()

    return pl.pallas_call(
        body,
        out_shape=jax.ShapeDtypeStruct((m_per, n), jnp.float32),
        in_specs=[pl.BlockSpec(memory_space=pltpu.VMEM)],
        out_specs=pl.BlockSpec(memory_space=pltpu.VMEM),
        scratch_shapes=[
            pltpu.VMEM((n_total, n), jnp.bfloat16),
            pltpu.SemaphoreType.DMA((N_DEV - 1,)),
            pltpu.SemaphoreType.DMA((N_DEV,)),
        ],
        compiler_params=pltpu.CompilerParams(collective_id=0),
    )(x)


# device time: 5663 ns/iter; 5.5492x vs baseline; 1.6532x over previous
import os

import jax
import jax.numpy as jnp
from jax import lax
from jax.experimental import pallas as pl
from jax.experimental.pallas import tpu as pltpu

N_DEV = 8
_ABLATE = os.environ.get("ABLATE", "")


def _pack(v):
    r = v.shape[0]
    return jnp.concatenate([v[: r // 2], v[r // 2 :]], axis=1)


def _unpack(v2):
    c = v2.shape[1] // 2
    return jnp.concatenate([v2[:, :c], v2[:, c:]], axis=0)


def _bitonic_stages_packed(v2, k_first, k_last, dir_asc=None):
    r2, c2 = v2.shape
    c = c2 // 2
    lane = lax.broadcasted_iota(jnp.int32, v2.shape, 1)
    row = lax.broadcasted_iota(jnp.int32, v2.shape, 0) + jnp.where(
        lane >= c, r2, 0
    )
    k = k_first
    while k <= k_last:
        asc = (row & k) == 0
        if dir_asc is not None:
            asc = asc == dir_asc
        j = k // 2
        while j >= 1:
            lower = (row & j) == 0
            if j == r2:
                partner = jnp.roll(v2, c, axis=1)
            else:
                down = jnp.roll(v2, -j, axis=0)
                up = jnp.roll(v2, j, axis=0)
                partner = jnp.where(lower, down, up)
            take_min = lower == asc
            v2 = jnp.where(take_min, jnp.minimum(v2, partner),
                           jnp.maximum(v2, partner))
            j //= 2
        k *= 2
    return v2


def kernel(x):
    m_per, n = x.shape
    n_total = N_DEV * m_per

    def body(x_ref, out_ref, gbuf_ref, send_sems, recv_sems):
        my = lax.axis_index("i")

        if _ABLATE != "nocomm":
            barrier_sem = pltpu.get_barrier_semaphore()
            for off in range(1, N_DEV):
                pl.semaphore_signal(
                    barrier_sem, inc=1,
                    device_id=((my + off) % N_DEV,),
                    device_id_type=pl.DeviceIdType.MESH,
                )

        dir_asc = (my % 2) == 0
        if _ABLATE in ("nosort", "nolocal"):
            gbuf_ref[pl.ds(my * m_per, m_per), :] = x_ref[:, :].astype(jnp.bfloat16)
        else:
            v_loc = _bitonic_stages_packed(
                _pack(x_ref[:, :].astype(jnp.bfloat16)), 2, m_per, dir_asc=dir_asc
            )
            gbuf_ref[pl.ds(my * m_per, m_per), :] = _unpack(v_loc)

        sends = []
        if _ABLATE != "nocomm":
            pl.semaphore_wait(barrier_sem, N_DEV - 1)

            my_slot = gbuf_ref.at[pl.ds(my * m_per, m_per), :]
            for off in range(1, N_DEV):
                rdma = pltpu.make_async_remote_copy(
                    src_ref=my_slot,
                    dst_ref=my_slot,
                    send_sem=send_sems.at[off - 1],
                    recv_sem=recv_sems.at[my],
                    device_id=((my + off) % N_DEV,),
                    device_id_type=pl.DeviceIdType.MESH,
                )
                rdma.start()
                sends.append(rdma)

            for off in range(1, N_DEV):
                src_dev = (my + off) % N_DEV
                slot = gbuf_ref.at[pl.ds(src_dev * m_per, m_per), :]
                recv = pltpu.make_async_remote_copy(
                    src_ref=slot,
                    dst_ref=slot,
                    send_sem=send_sems.at[off - 1],
                    recv_sem=recv_sems.at[src_dev],
                    device_id=(src_dev,),
                    device_id_type=pl.DeviceIdType.MESH,
                )
                recv.wait_recv()

        if _ABLATE not in ("nosort", "nomerge"):
            v2 = _bitonic_stages_packed(_pack(gbuf_ref[:, :]), 2 * m_per, n_total)
            gbuf_ref[:, :] = _unpack(v2)
        out_ref[:, :] = gbuf_ref[pl.ds(my * m_per, m_per), :].astype(jnp.float32)

        for rdma in sends:
            rdma.wait_send()

    return pl.pallas_call(
        body,
        out_shape=jax.ShapeDtypeStruct((m_per, n), jnp.float32),
        in_specs=[pl.BlockSpec(memory_space=pltpu.VMEM)],
        out_specs=pl.BlockSpec(memory_space=pltpu.VMEM),
        scratch_shapes=[
            pltpu.VMEM((n_total, n), jnp.bfloat16),
            pltpu.SemaphoreType.DMA((N_DEV - 1,)),
            pltpu.SemaphoreType.DMA((N_DEV,)),
        ],
        compiler_params=(
            None if _ABLATE == "nocomm"
            else pltpu.CompilerParams(collective_id=0)
        ),
    )(x)
